# baseline (speedup 1.0000x reference)
import jax
import jax.numpy as jnp
from jax import lax
from jax.experimental import pallas as pl
from jax.experimental.pallas import tpu as pltpu
from jax.experimental.pallas import tpu_sc as plsc

VOCAB_SIZE = 1_000_000
EMBED_DIM = 32
BATCH = 16384
HIST = 50
HIST_PAD = 56
LANE_PAD = 128
Q_TOTAL = BATCH * HIST_PAD
NUM_WORKERS = 32
Q_PER_W = Q_TOTAL // NUM_WORKERS
CQ = 1792
NCHUNK = Q_PER_W // CQ


def _body(idx_hbm, table_hbm, out_hbm, idx0, idx1, rows0, rows1, sem0, sem1):
    wid = lax.axis_index("s") * 2 + lax.axis_index("c")
    base = wid * Q_PER_W

    def _wait(rows_v, sem):
        pltpu.make_async_copy(
            table_hbm.at[pl.ds(0, CQ)], rows_v, sem
        ).wait()

    def _start(c, idx_v, rows_v, sem):
        pltpu.sync_copy(idx_hbm.at[pl.ds(base + c * CQ, CQ)], idx_v)
        pltpu.async_copy(table_hbm.at[idx_v], rows_v, sem)

    def _store(c, rows_v):
        pltpu.sync_copy(
            rows_v, out_hbm.at[pl.ds(base + c * CQ, CQ), pl.ds(0, EMBED_DIM)]
        )

    _start(0, idx0, rows0, sem0)

    @pl.loop(0, NCHUNK, step=2)
    def _chunk(c):
        _start(c + 1, idx1, rows1, sem1)
        _wait(rows0, sem0)
        _store(c, rows0)

        @pl.when(c + 2 < NCHUNK)
        def _():
            _start(c + 2, idx0, rows0, sem0)

        _wait(rows1, sem1)
        _store(c + 1, rows1)


@jax.jit
def _embed(token_ids, embeddings):
    idx56 = jnp.concatenate(
        [token_ids, token_ids[:, : HIST_PAD - HIST]], axis=1
    ).astype(jnp.int32)
    idx_flat = idx56.reshape(-1)

    mesh = plsc.VectorSubcoreMesh(core_axis_name="c", subcore_axis_name="s")
    grid_kernel = pl.kernel(
        _body,
        out_type=jax.ShapeDtypeStruct((Q_TOTAL, LANE_PAD), jnp.float32),
        mesh=mesh,
        scratch_types=[
            pltpu.VMEM((CQ,), jnp.int32),
            pltpu.VMEM((CQ,), jnp.int32),
            pltpu.VMEM((CQ, EMBED_DIM), jnp.float32),
            pltpu.VMEM((CQ, EMBED_DIM), jnp.float32),
            pltpu.SemaphoreType.DMA,
            pltpu.SemaphoreType.DMA,
        ],
        compiler_params=pltpu.CompilerParams(use_tc_tiling_on_sc=False),
    )
    padded = grid_kernel(idx_flat, embeddings)
    padded3 = padded.reshape(BATCH, HIST_PAD, LANE_PAD)
    return lax.slice(padded3, (0, 0, 0), (BATCH, HIST, EMBED_DIM))


def kernel(token_ids, embeddings):
    return _embed(token_ids, embeddings)
